# baseline (speedup 1.0000x reference)
import functools

import jax
import jax.numpy as jnp
from jax import lax
from jax.experimental import pallas as pl
from jax.experimental.pallas import tpu as pltpu
from jax.experimental.pallas import tpu_sc as plsc

NU = 50000
NI = 50000
NN = NU + NI
D = 32
H = 16
N_LAYERS = 3
E = 1600000
B = 4096
EPS = 1e-5
LOSS_W = 0.1

NC, NS = 2, 16
CHUNK = 512
SUB = 128
NSUB = CHUNK // SUB
EPT = 100352
NCHUNKS = EPT // CHUNK
NV_LAST = (E - (NS - 1) * EPT) // CHUNK
NIO = 4
NN_PAD = 100096
ROWS_PT = NN_PAD // NS
ZCOPIES = ROWS_PT // CHUNK
ZTAIL = ROWS_PT - ZCOPIES * CHUNK


def _seg_body(emb2, ei3, val, out,
              col_v, row_v, val_v, gath_v, acc, isem, gsemA, gsemB, ssem):
    c = lax.axis_index("c")
    s = lax.axis_index("s")
    cNN = c * NN
    gsems = (gsemA, gsemB)
    nvalid = jnp.where(s == NS - 1, NV_LAST, NCHUNKS)

    @plsc.parallel_loop(0, CHUNK, unroll=4)
    def _z(i):
        gath_v[0, i, :] = jnp.zeros((H,), jnp.float32)

    zcopies = [pltpu.async_copy(
        gath_v.at[0], acc.at[pl.ds(s * ROWS_PT + t * CHUNK, CHUNK)], isem)
        for t in range(ZCOPIES)]
    zcopies.append(pltpu.async_copy(
        gath_v.at[0, pl.ds(0, ZTAIL)],
        acc.at[pl.ds(s * ROWS_PT + ZCOPIES * CHUNK, ZTAIL)], isem))
    for z in zcopies:
        z.wait()
    plsc.subcore_barrier()

    def in_descs(m, q):
        r0 = s * (EPT // SUB) + m * NSUB
        e0 = s * EPT + m * CHUNK
        return ((ei3.at[1, pl.ds(r0, NSUB)], col_v.at[q]),
                (ei3.at[0, pl.ds(r0, NSUB)], row_v.at[q]),
                (val.at[pl.ds(e0, CHUNK)], val_v.at[q]))

    def fire_in(m, q):
        for src, dst in in_descs(m, q):
            pltpu.async_copy(src, dst, isem)

    def wait_in(m, q):
        for src, dst in in_descs(m, q):
            pltpu.make_async_copy(src, dst, isem).wait()

    def adjust(q):
        for j in range(NSUB):
            @plsc.parallel_loop(0, SUB // 16)
            def _adj(l):
                sl = pl.ds(l * 16, 16)
                col_v[q, j, sl] = col_v[q, j, sl] + cNN

    def gather_descs(p, q):
        return tuple((emb2.at[col_v.at[q, j]],
                      gath_v.at[p, pl.ds(j * SUB, SUB)], gsems[p])
                     for j in range(NSUB))

    def scatter_descs(p, q):
        return tuple((gath_v.at[p, pl.ds(j * SUB, SUB)],
                      acc.at[row_v.at[q, j]]) for j in range(NSUB))

    def scale(p, q):
        @plsc.parallel_loop(0, CHUNK // 16, unroll=2)
        def _scale(g):
            wv = val_v[q, pl.ds(g * 16, 16)]
            e0g = g * 16
            for l in range(16):
                gath_v[p, e0g + l, :] = gath_v[p, e0g + l, :] * wv[l]

    def phase(k, p, q):
        @pl.when((k > 0) & (k <= nvalid))
        def _():
            for src, dst in scatter_descs(1 - p, (q - 1) % NIO):
                pltpu.make_async_copy(src, dst, ssem).wait()

        @pl.when(k + 1 < nvalid)
        def _():
            wait_in(k + 1, (q + 1) % NIO)
            adjust((q + 1) % NIO)
            for src, dst, sem in gather_descs(1 - p, (q + 1) % NIO):
                pltpu.async_copy(src, dst, sem)

        @pl.when(k + 2 < nvalid)
        def _():
            fire_in(k + 2, (q + 2) % NIO)

        @pl.when(k < nvalid)
        def _():
            for src, dst, sem in gather_descs(p, q):
                pltpu.make_async_copy(src, dst, sem).wait()
            scale(p, q)
            for src, dst in scatter_descs(p, q):
                pltpu.async_copy(src, dst, ssem, add=True)

    fire_in(0, 0)
    wait_in(0, 0)
    adjust(0)
    for src, dst, sem in gather_descs(0, 0):
        pltpu.async_copy(src, dst, sem)
    fire_in(1, 1)

    @pl.loop(0, NCHUNKS // NIO)
    def chunk_body(k6):
        k0 = k6 * NIO
        for ph in range(NIO):
            phase(k0 + ph, ph % 2, ph)

    @pl.when(s < NS - 1)
    def _():
        for src, dst in scatter_descs((NCHUNKS - 1) % 2, (NCHUNKS - 1) % NIO):
            pltpu.make_async_copy(src, dst, ssem).wait()

    plsc.subcore_barrier()

    r0 = s * ROWS_PT
    pltpu.sync_copy(acc.at[pl.ds(r0, ROWS_PT)], out.at[c, pl.ds(r0, ROWS_PT)])


@functools.cache
def _seg_call():
    return pl.kernel(
        _seg_body,
        out_type=jax.ShapeDtypeStruct((NC, NN_PAD, H), jnp.float32),
        mesh=plsc.VectorSubcoreMesh(core_axis_name="c", subcore_axis_name="s",
                                    num_cores=NC, num_subcores=NS),
        scratch_types=[
            pltpu.VMEM((NIO, NSUB, SUB), jnp.int32),
            pltpu.VMEM((NIO, NSUB, SUB), jnp.int32),
            pltpu.VMEM((NIO, CHUNK), jnp.float32),
            pltpu.VMEM((2, CHUNK, H), jnp.float32),
            pltpu.VMEM_SHARED((NN_PAD, H), jnp.float32),
            pltpu.SemaphoreType.DMA,
            pltpu.SemaphoreType.DMA,
            pltpu.SemaphoreType.DMA,
            pltpu.SemaphoreType.DMA,
        ],
        compiler_params=pltpu.CompilerParams(use_tc_tiling_on_sc=False),
    )


_B_PW = B // (NC * NS)


def _lookup_body(e0, e1, e2, e3, users, pos, neg, u_out, p_out, n_out,
                 idx_v, jdx_v, r0a, r0b, r1a, r1b, r2a, r2b, r3a, r3b, sem):
    w = lax.axis_index("s") * NC + lax.axis_index("c")
    base = w * _B_PW
    plane0 = (r0a, r1a, r2a, r3a)
    plane1 = (r0b, r1b, r2b, r3b)

    for off, src, dst in ((0, users, u_out), (NU, pos, p_out), (NU, neg, n_out)):
        pltpu.sync_copy(src.at[pl.ds(base, _B_PW)], idx_v)

        @plsc.parallel_loop(0, _B_PW // 16)
        def _addl(l):
            sl = pl.ds(l * 16, 16)
            idx_v[sl] = idx_v[sl] + off
            jdx_v[sl] = idx_v[sl] + NN

        copies = [pltpu.async_copy(t.at[idx_v], r, sem)
                  for t, r in zip((e0, e1, e2, e3), plane0)]
        copies += [pltpu.async_copy(t.at[jdx_v], r, sem)
                   for t, r in zip((e0, e1, e2, e3), plane1)]
        for cp in copies:
            cp.wait()

        @plsc.parallel_loop(0, _B_PW)
        def _avg(r):
            r0a[r, :] = (r0a[r, :] + r1a[r, :] + r2a[r, :] + r3a[r, :]) * 0.25
            r0b[r, :] = (r0b[r, :] + r1b[r, :] + r2b[r, :] + r3b[r, :]) * 0.25

        pltpu.sync_copy(r0a, dst.at[pl.ds(base, _B_PW), pl.ds(0, H)])
        pltpu.sync_copy(r0b, dst.at[pl.ds(base, _B_PW), pl.ds(H, H)])


@functools.cache
def _lookup_call():
    return pl.kernel(
        _lookup_body,
        out_type=(jax.ShapeDtypeStruct((B, D), jnp.float32),) * 3,
        mesh=plsc.VectorSubcoreMesh(core_axis_name="c", subcore_axis_name="s",
                                    num_cores=NC, num_subcores=NS),
        scratch_types=[
            pltpu.VMEM((_B_PW,), jnp.int32),
            pltpu.VMEM((_B_PW,), jnp.int32),
        ] + [pltpu.VMEM((_B_PW, H), jnp.float32)] * 8 + [
            pltpu.SemaphoreType.DMA,
        ],
        compiler_params=pltpu.CompilerParams(use_tc_tiling_on_sc=False),
    )


_RB = 2000
_NUB = NU // _RB


def _prep_body(ut_ref, cf_ref, wp_ref, bp_ref, wg_ref, bg_ref, it_ref,
               emb_ref, proj_ref):
    i = pl.program_id(0)

    @pl.when(i < _NUB)
    def _():
        x = ut_ref[...]
        emb_ref[0] = x[:, :H]
        emb_ref[1] = x[:, H:]

    @pl.when(i >= _NUB)
    def _():
        cf = cf_ref[...]
        proj = jnp.dot(cf, wp_ref[...], preferred_element_type=jnp.float32,
                       precision=lax.Precision.HIGHEST) + bp_ref[...]
        logits = jnp.sum(cf * wg_ref[...], axis=1, keepdims=True) + bg_ref[0, 0]
        g = jax.nn.sigmoid(logits)
        ie = (1.0 - g) * it_ref[...] + g * proj
        emb_ref[0] = ie[:, :H]
        emb_ref[1] = ie[:, H:]
        proj_ref[...] = proj


def _prep_call(user_table, content, W_proj, b_proj, W_gate, b_gate,
               item_table):
    return pl.pallas_call(
        _prep_body,
        grid=(NN // _RB,),
        in_specs=[
            pl.BlockSpec((_RB, D), lambda i: (jnp.minimum(i, _NUB - 1), 0)),
            pl.BlockSpec((_RB, 256), lambda i: (jnp.maximum(i - _NUB, 0), 0)),
            pl.BlockSpec((256, D), lambda i: (0, 0)),
            pl.BlockSpec((1, D), lambda i: (0, 0)),
            pl.BlockSpec((1, 256), lambda i: (0, 0)),
            pl.BlockSpec((1, 1), lambda i: (0, 0), memory_space=pltpu.SMEM),
            pl.BlockSpec((_RB, D), lambda i: (jnp.maximum(i - _NUB, 0), 0)),
        ],
        out_specs=[
            pl.BlockSpec((NC, _RB, H), lambda i: (0, i, 0)),
            pl.BlockSpec((_RB, D), lambda i: (jnp.maximum(i - _NUB, 0), 0)),
        ],
        out_shape=[
            jax.ShapeDtypeStruct((NC, NN, H), jnp.float32),
            jax.ShapeDtypeStruct((NI, D), jnp.float32),
        ],
    )(user_table, content, W_proj, b_proj.reshape(1, D),
      W_gate.reshape(1, 256), b_gate.reshape(1, 1), item_table)


def _ln_body(seg_ref, prev_ref, out_ref):
    s0 = seg_ref[0]
    s1 = seg_ref[1]
    m = (jnp.sum(s0, axis=1, keepdims=True) +
         jnp.sum(s1, axis=1, keepdims=True)) * (1.0 / D)
    d0 = s0 - m
    d1 = s1 - m
    v = (jnp.sum(d0 * d0, axis=1, keepdims=True) +
         jnp.sum(d1 * d1, axis=1, keepdims=True)) * (1.0 / D)
    inv = lax.rsqrt(v + EPS)
    out_ref[0] = d0 * inv + prev_ref[0]
    out_ref[1] = d1 * inv + prev_ref[1]


def _ln_call(seg, prev):
    sspec = pl.BlockSpec((NC, _RB, H), lambda i: (0, i, 0))
    return pl.pallas_call(
        _ln_body,
        grid=(NN // _RB,),
        in_specs=[sspec, sspec],
        out_specs=sspec,
        out_shape=jax.ShapeDtypeStruct((NC, NN, H), jnp.float32),
    )(seg, prev)


def _final_body(e0_ref, e1_ref, e2_ref, e3_ref, proj_ref, loss_ref):
    i = pl.program_id(0)
    h0 = (e0_ref[0] + e1_ref[0] + e2_ref[0] + e3_ref[0]) * 0.25
    h1 = (e0_ref[1] + e1_ref[1] + e2_ref[1] + e3_ref[1]) * 0.25
    pr = proj_ref[...]
    dd0 = h0 - pr[:, :H]
    dd1 = h1 - pr[:, H:]

    @pl.when(i == 0)
    def _():
        loss_ref[0, 0] = 0.0

    loss_ref[0, 0] += jnp.sum(dd0 * dd0) + jnp.sum(dd1 * dd1)


def _final_call(e0, e1, e2, e3, proj):
    ispec = pl.BlockSpec((NC, _RB, H), lambda i: (0, i + _NUB, 0))
    return pl.pallas_call(
        _final_body,
        grid=(NI // _RB,),
        in_specs=[ispec, ispec, ispec, ispec,
                  pl.BlockSpec((_RB, D), lambda i: (i, 0))],
        out_specs=pl.BlockSpec((1, 1), lambda i: (0, 0),
                               memory_space=pltpu.SMEM),
        out_shape=jax.ShapeDtypeStruct((1, 1), jnp.float32),
    )(e0, e1, e2, e3, proj)


def kernel(users, pos_items, neg_items, edge_index, graph_values,
           content_features, user_table, item_table, W_proj, b_proj,
           W_gate, b_gate):
    users = users.astype(jnp.int32)
    pos_items = pos_items.astype(jnp.int32)
    neg_items = neg_items.astype(jnp.int32)
    ei3 = edge_index.astype(jnp.int32).reshape(2, E // SUB, SUB)

    emb, proj = _prep_call(user_table, content_features, W_proj, b_proj,
                           W_gate, b_gate, item_table)

    embs = [emb]
    for _ in range(N_LAYERS):
        seg = _seg_call()(emb.reshape(NC * NN, H), ei3, graph_values)
        emb = _ln_call(seg, emb)
        embs.append(emb)

    loss_sum = _final_call(embs[0], embs[1], embs[2], embs[3], proj)
    users_emb, pos_emb, neg_emb = _lookup_call()(
        embs[0].reshape(NC * NN, H), embs[1].reshape(NC * NN, H),
        embs[2].reshape(NC * NN, H), embs[3].reshape(NC * NN, H),
        users, pos_items, neg_items)
    content_loss = loss_sum[0, 0] * (LOSS_W / (NI * D))
    return (users_emb, pos_emb, neg_emb, content_loss)


# speedup vs baseline: 1.0210x; 1.0210x over previous
import functools

import jax
import jax.numpy as jnp
from jax import lax
from jax.experimental import pallas as pl
from jax.experimental.pallas import tpu as pltpu
from jax.experimental.pallas import tpu_sc as plsc

NU = 50000
NI = 50000
NN = NU + NI
D = 32
H = 16
N_LAYERS = 3
E = 1600000
B = 4096
EPS = 1e-5
LOSS_W = 0.1

NC, NS = 2, 16
CHUNK = 512
SUB = 128
NSUB = CHUNK // SUB
EPT = 100352
NCHUNKS = EPT // CHUNK
NV_LAST = (E - (NS - 1) * EPT) // CHUNK
NIO = 4
NN_PAD = 100096
ROWS_PT = NN_PAD // NS
ZCOPIES = ROWS_PT // CHUNK
ZTAIL = ROWS_PT - ZCOPIES * CHUNK


def _seg_body(emb3, ei3, val, out,
              col_v, row_v, val_v, gath_v, acc, isem, gsemA, gsemB, ssem):
    c = lax.axis_index("c")
    s = lax.axis_index("s")
    emb_pl = emb3.at[c]
    gsems = (gsemA, gsemB)
    nvalid = jnp.where(s == NS - 1, NV_LAST, NCHUNKS)

    @plsc.parallel_loop(0, CHUNK, unroll=4)
    def _z(i):
        gath_v[0, i, :] = jnp.zeros((H,), jnp.float32)

    zcopies = [pltpu.async_copy(
        gath_v.at[0], acc.at[pl.ds(s * ROWS_PT + t * CHUNK, CHUNK)], isem)
        for t in range(ZCOPIES)]
    zcopies.append(pltpu.async_copy(
        gath_v.at[0, pl.ds(0, ZTAIL)],
        acc.at[pl.ds(s * ROWS_PT + ZCOPIES * CHUNK, ZTAIL)], isem))
    for z in zcopies:
        z.wait()
    plsc.subcore_barrier()

    def in_descs(m, q):
        r0 = s * (EPT // SUB) + m * NSUB
        e0 = s * EPT + m * CHUNK
        return ((ei3.at[1, pl.ds(r0, NSUB)], col_v.at[q]),
                (ei3.at[0, pl.ds(r0, NSUB)], row_v.at[q]),
                (val.at[pl.ds(e0, CHUNK)], val_v.at[q]))

    def fire_in(m, q):
        for src, dst in in_descs(m, q):
            pltpu.async_copy(src, dst, isem)

    def wait_in(m, q):
        for src, dst in in_descs(m, q):
            pltpu.make_async_copy(src, dst, isem).wait()

    def gather_descs(p, q):
        return tuple((emb_pl.at[col_v.at[q, j]],
                      gath_v.at[p, pl.ds(j * SUB, SUB)], gsems[p])
                     for j in range(NSUB))

    def scatter_descs(p, q):
        return tuple((gath_v.at[p, pl.ds(j * SUB, SUB)],
                      acc.at[row_v.at[q, j]]) for j in range(NSUB))

    def scale(p, q):
        @plsc.parallel_loop(0, CHUNK // 16, unroll=2)
        def _scale(g):
            wv = val_v[q, pl.ds(g * 16, 16)]
            e0g = g * 16
            for l in range(16):
                gath_v[p, e0g + l, :] = gath_v[p, e0g + l, :] * wv[l]

    def phase(k, p, q):
        @pl.when((k > 0) & (k <= nvalid))
        def _():
            for src, dst in scatter_descs(1 - p, (q - 1) % NIO):
                pltpu.make_async_copy(src, dst, ssem).wait()

        @pl.when(k + 1 < nvalid)
        def _():
            wait_in(k + 1, (q + 1) % NIO)
            for src, dst, sem in gather_descs(1 - p, (q + 1) % NIO):
                pltpu.async_copy(src, dst, sem)

        @pl.when(k + 2 < nvalid)
        def _():
            fire_in(k + 2, (q + 2) % NIO)

        @pl.when(k < nvalid)
        def _():
            for src, dst, sem in gather_descs(p, q):
                pltpu.make_async_copy(src, dst, sem).wait()
            scale(p, q)
            for src, dst in scatter_descs(p, q):
                pltpu.async_copy(src, dst, ssem, add=True)

    fire_in(0, 0)
    wait_in(0, 0)
    for src, dst, sem in gather_descs(0, 0):
        pltpu.async_copy(src, dst, sem)
    fire_in(1, 1)

    @pl.loop(0, NCHUNKS // NIO)
    def chunk_body(k6):
        k0 = k6 * NIO
        for ph in range(NIO):
            phase(k0 + ph, ph % 2, ph)

    @pl.when(s < NS - 1)
    def _():
        for src, dst in scatter_descs((NCHUNKS - 1) % 2, (NCHUNKS - 1) % NIO):
            pltpu.make_async_copy(src, dst, ssem).wait()

    plsc.subcore_barrier()

    r0 = s * ROWS_PT
    pltpu.sync_copy(acc.at[pl.ds(r0, ROWS_PT)], out.at[c, pl.ds(r0, ROWS_PT)])


@functools.cache
def _seg_call():
    return pl.kernel(
        _seg_body,
        out_type=jax.ShapeDtypeStruct((NC, NN_PAD, H), jnp.float32),
        mesh=plsc.VectorSubcoreMesh(core_axis_name="c", subcore_axis_name="s",
                                    num_cores=NC, num_subcores=NS),
        scratch_types=[
            pltpu.VMEM((NIO, NSUB, SUB), jnp.int32),
            pltpu.VMEM((NIO, NSUB, SUB), jnp.int32),
            pltpu.VMEM((NIO, CHUNK), jnp.float32),
            pltpu.VMEM((2, CHUNK, H), jnp.float32),
            pltpu.VMEM_SHARED((NN_PAD, H), jnp.float32),
            pltpu.SemaphoreType.DMA,
            pltpu.SemaphoreType.DMA,
            pltpu.SemaphoreType.DMA,
            pltpu.SemaphoreType.DMA,
        ],
        compiler_params=pltpu.CompilerParams(use_tc_tiling_on_sc=False),
    )


_B_PW = B // (NC * NS)


def _lookup_body(e0, e1, e2, e3, users, pos, neg, u_out, p_out, n_out,
                 idx_v, r0a, r0b, r1a, r1b, r2a, r2b, r3a, r3b, sem):
    w = lax.axis_index("s") * NC + lax.axis_index("c")
    base = w * _B_PW
    plane0 = (r0a, r1a, r2a, r3a)
    plane1 = (r0b, r1b, r2b, r3b)

    for off, src, dst in ((0, users, u_out), (NU, pos, p_out), (NU, neg, n_out)):
        pltpu.sync_copy(src.at[pl.ds(base, _B_PW)], idx_v)
        if off:
            @plsc.parallel_loop(0, _B_PW // 16)
            def _addl(l):
                sl = pl.ds(l * 16, 16)
                idx_v[sl] = idx_v[sl] + off

        copies = [pltpu.async_copy(t.at[0].at[idx_v], r, sem)
                  for t, r in zip((e0, e1, e2, e3), plane0)]
        copies += [pltpu.async_copy(t.at[1].at[idx_v], r, sem)
                   for t, r in zip((e0, e1, e2, e3), plane1)]
        for cp in copies:
            cp.wait()

        @plsc.parallel_loop(0, _B_PW)
        def _avg(r):
            r0a[r, :] = (r0a[r, :] + r1a[r, :] + r2a[r, :] + r3a[r, :]) * 0.25
            r0b[r, :] = (r0b[r, :] + r1b[r, :] + r2b[r, :] + r3b[r, :]) * 0.25

        pltpu.sync_copy(r0a, dst.at[pl.ds(base, _B_PW), pl.ds(0, H)])
        pltpu.sync_copy(r0b, dst.at[pl.ds(base, _B_PW), pl.ds(H, H)])


@functools.cache
def _lookup_call():
    return pl.kernel(
        _lookup_body,
        out_type=(jax.ShapeDtypeStruct((B, D), jnp.float32),) * 3,
        mesh=plsc.VectorSubcoreMesh(core_axis_name="c", subcore_axis_name="s",
                                    num_cores=NC, num_subcores=NS),
        scratch_types=[
            pltpu.VMEM((_B_PW,), jnp.int32),
        ] + [pltpu.VMEM((_B_PW, H), jnp.float32)] * 8 + [
            pltpu.SemaphoreType.DMA,
        ],
        compiler_params=pltpu.CompilerParams(use_tc_tiling_on_sc=False),
    )


_RB = 2000
_NUB = NU // _RB


def _prep_body(ut_ref, cf_ref, wp_ref, bp_ref, wg_ref, bg_ref, it_ref,
               emb_ref, proj_ref):
    i = pl.program_id(0)

    @pl.when(i < _NUB)
    def _():
        x = ut_ref[...]
        emb_ref[0] = x[:, :H]
        emb_ref[1] = x[:, H:]

    @pl.when(i >= _NUB)
    def _():
        cf = cf_ref[...]
        proj = jnp.dot(cf, wp_ref[...], preferred_element_type=jnp.float32,
                       precision=lax.Precision.HIGHEST) + bp_ref[...]
        logits = jnp.sum(cf * wg_ref[...], axis=1, keepdims=True) + bg_ref[0, 0]
        g = jax.nn.sigmoid(logits)
        ie = (1.0 - g) * it_ref[...] + g * proj
        emb_ref[0] = ie[:, :H]
        emb_ref[1] = ie[:, H:]
        proj_ref[...] = proj


def _prep_call(user_table, content, W_proj, b_proj, W_gate, b_gate,
               item_table):
    return pl.pallas_call(
        _prep_body,
        grid=(NN // _RB,),
        in_specs=[
            pl.BlockSpec((_RB, D), lambda i: (jnp.minimum(i, _NUB - 1), 0)),
            pl.BlockSpec((_RB, 256), lambda i: (jnp.maximum(i - _NUB, 0), 0)),
            pl.BlockSpec((256, D), lambda i: (0, 0)),
            pl.BlockSpec((1, D), lambda i: (0, 0)),
            pl.BlockSpec((1, 256), lambda i: (0, 0)),
            pl.BlockSpec((1, 1), lambda i: (0, 0), memory_space=pltpu.SMEM),
            pl.BlockSpec((_RB, D), lambda i: (jnp.maximum(i - _NUB, 0), 0)),
        ],
        out_specs=[
            pl.BlockSpec((NC, _RB, H), lambda i: (0, i, 0)),
            pl.BlockSpec((_RB, D), lambda i: (jnp.maximum(i - _NUB, 0), 0)),
        ],
        out_shape=[
            jax.ShapeDtypeStruct((NC, NN, H), jnp.float32),
            jax.ShapeDtypeStruct((NI, D), jnp.float32),
        ],
    )(user_table, content, W_proj, b_proj.reshape(1, D),
      W_gate.reshape(1, 256), b_gate.reshape(1, 1), item_table)


def _ln_body(seg_ref, prev_ref, out_ref):
    s0 = seg_ref[0]
    s1 = seg_ref[1]
    m = (jnp.sum(s0, axis=1, keepdims=True) +
         jnp.sum(s1, axis=1, keepdims=True)) * (1.0 / D)
    d0 = s0 - m
    d1 = s1 - m
    v = (jnp.sum(d0 * d0, axis=1, keepdims=True) +
         jnp.sum(d1 * d1, axis=1, keepdims=True)) * (1.0 / D)
    inv = lax.rsqrt(v + EPS)
    out_ref[0] = d0 * inv + prev_ref[0]
    out_ref[1] = d1 * inv + prev_ref[1]


def _ln_call(seg, prev):
    sspec = pl.BlockSpec((NC, _RB, H), lambda i: (0, i, 0))
    return pl.pallas_call(
        _ln_body,
        grid=(NN // _RB,),
        in_specs=[sspec, sspec],
        out_specs=sspec,
        out_shape=jax.ShapeDtypeStruct((NC, NN, H), jnp.float32),
    )(seg, prev)


def _final_body(e0_ref, e1_ref, e2_ref, e3_ref, proj_ref, loss_ref):
    i = pl.program_id(0)
    h0 = (e0_ref[0] + e1_ref[0] + e2_ref[0] + e3_ref[0]) * 0.25
    h1 = (e0_ref[1] + e1_ref[1] + e2_ref[1] + e3_ref[1]) * 0.25
    pr = proj_ref[...]
    dd0 = h0 - pr[:, :H]
    dd1 = h1 - pr[:, H:]

    @pl.when(i == 0)
    def _():
        loss_ref[0, 0] = 0.0

    loss_ref[0, 0] += jnp.sum(dd0 * dd0) + jnp.sum(dd1 * dd1)


def _final_call(e0, e1, e2, e3, proj):
    ispec = pl.BlockSpec((NC, _RB, H), lambda i: (0, i + _NUB, 0))
    return pl.pallas_call(
        _final_body,
        grid=(NI // _RB,),
        in_specs=[ispec, ispec, ispec, ispec,
                  pl.BlockSpec((_RB, D), lambda i: (i, 0))],
        out_specs=pl.BlockSpec((1, 1), lambda i: (0, 0),
                               memory_space=pltpu.SMEM),
        out_shape=jax.ShapeDtypeStruct((1, 1), jnp.float32),
    )(e0, e1, e2, e3, proj)


def kernel(users, pos_items, neg_items, edge_index, graph_values,
           content_features, user_table, item_table, W_proj, b_proj,
           W_gate, b_gate):
    users = users.astype(jnp.int32)
    pos_items = pos_items.astype(jnp.int32)
    neg_items = neg_items.astype(jnp.int32)
    ei3 = edge_index.astype(jnp.int32).reshape(2, E // SUB, SUB)

    emb, proj = _prep_call(user_table, content_features, W_proj, b_proj,
                           W_gate, b_gate, item_table)

    embs = [emb]
    for _ in range(N_LAYERS):
        seg = _seg_call()(emb, ei3, graph_values)
        emb = _ln_call(seg, emb)
        embs.append(emb)

    loss_sum = _final_call(embs[0], embs[1], embs[2], embs[3], proj)
    users_emb, pos_emb, neg_emb = _lookup_call()(
        embs[0], embs[1], embs[2], embs[3], users, pos_items, neg_items)
    content_loss = loss_sum[0, 0] * (LOSS_W / (NI * D))
    return (users_emb, pos_emb, neg_emb, content_loss)
